# baseline (speedup 1.0000x reference)
import jax
import jax.numpy as jnp
from jax import lax
from jax.experimental import pallas as pl
from jax.experimental.pallas import tpu as pltpu
from jax.experimental.pallas import tpu_sc as plsc

EMBED = 64
B = 16384
NEG = 64

_NC = 2
_NS = 16
_NW = _NC * _NS
_BPW = B // _NW
_CH = 128
_K = _BPW // _CH

_G = 8
_BLK = B // _G
_WPG = _BLK // _BPW


def _extract_column(tri_v, col, idx_v):
    lanes = lax.iota(jnp.int32, 16)
    cols = jnp.full((16,), col, jnp.int32)

    def step(k, _):
        rows = lanes + k * 16
        vals = plsc.load_gather(tri_v, [rows, cols])
        idx_v[pl.ds(k * 16, 16)] = vals
        return ()

    lax.fori_loop(0, _BPW // 16, step, ())


def _sc_user_body(tri_hbm, ut_hbm, u_out, tri_v, idx_v, rows_u, sem_u):
    c = lax.axis_index("c")
    s = lax.axis_index("s")
    wid = s * _NC + c
    base = wid * _BPW

    pltpu.sync_copy(tri_hbm.at[pl.ds(base, _BPW), :], tri_v)
    _extract_column(tri_v, 0, idx_v)
    cps = [
        pltpu.async_copy(
            ut_hbm.at[idx_v.at[pl.ds(j * _CH, _CH)]],
            rows_u.at[pl.ds(j * _CH, _CH)], sem_u)
        for j in range(_K)
    ]
    for cp in cps:
        cp.wait()
    pltpu.sync_copy(rows_u, u_out.at[pl.ds(base, _BPW)])


def _sc_user_gather(batch_triples, user_table):
    mesh = plsc.VectorSubcoreMesh(core_axis_name="c", subcore_axis_name="s")
    return pl.kernel(
        _sc_user_body,
        out_type=jax.ShapeDtypeStruct((B, EMBED), jnp.float32),
        mesh=mesh,
        scratch_types=[
            pltpu.VMEM((_BPW, 3), jnp.int32),
            pltpu.VMEM((_BPW,), jnp.int32),
            pltpu.VMEM((_BPW, EMBED), jnp.float32),
            pltpu.SemaphoreType.DMA,
        ],
        compiler_params=pltpu.CompilerParams(use_tc_tiling_on_sc=False,
                                             needs_layout_passes=False),
    )(batch_triples, user_table)


def _sc_prod_body(tri_hbm, pt_hbm, pb_hbm, ni_hbm,
                  t_out, b_out, n_out,
                  tri_v, idx_v, rows_t, bias_v, nidx, nrows,
                  sem_t, sem_b, sem_n):
    c = lax.axis_index("c")
    s = lax.axis_index("s")
    wid = s * _NC + c
    base = wid * _BPW

    pltpu.sync_copy(tri_hbm.at[pl.ds(base, _BPW), :], tri_v)
    _extract_column(tri_v, 2, idx_v)
    cps = []
    for j in range(_K):
        ids = idx_v.at[pl.ds(j * _CH, _CH)]
        cps.append(pltpu.async_copy(
            pt_hbm.at[ids], rows_t.at[pl.ds(j * _CH, _CH)], sem_t))
        cps.append(pltpu.async_copy(
            pb_hbm.at[ids], bias_v.at[pl.ds(j * _CH, _CH)], sem_b))

    @pl.when(wid == 0)
    def _():
        pltpu.sync_copy(ni_hbm, nidx)
        pltpu.async_copy(pt_hbm.at[nidx], nrows, sem_n).wait()
        pltpu.sync_copy(nrows, n_out)

    for cp in cps:
        cp.wait()
    pltpu.sync_copy(rows_t, t_out.at[pl.ds(base, _BPW)])
    pltpu.sync_copy(bias_v, b_out.at[pl.ds(base, _BPW)])


def _sc_prod_gather(batch_triples, product_table, purchase_bias, neg_idxs):
    mesh = plsc.VectorSubcoreMesh(core_axis_name="c", subcore_axis_name="s")
    return pl.kernel(
        _sc_prod_body,
        out_type=(
            jax.ShapeDtypeStruct((B, EMBED), jnp.float32),
            jax.ShapeDtypeStruct((B, 1), jnp.float32),
            jax.ShapeDtypeStruct((NEG, EMBED), jnp.float32),
        ),
        mesh=mesh,
        scratch_types=[
            pltpu.VMEM((_BPW, 3), jnp.int32),
            pltpu.VMEM((_BPW,), jnp.int32),
            pltpu.VMEM((_BPW, EMBED), jnp.float32),
            pltpu.VMEM((_BPW, 1), jnp.float32),
            pltpu.VMEM((NEG,), jnp.int32),
            pltpu.VMEM((NEG, EMBED), jnp.float32),
            pltpu.SemaphoreType.DMA,
            pltpu.SemaphoreType.DMA,
            pltpu.SemaphoreType.DMA,
        ],
        compiler_params=pltpu.CompilerParams(use_tc_tiling_on_sc=False,
                                             needs_layout_passes=False),
    )(batch_triples, product_table, purchase_bias, neg_idxs)


def _softplus(x):
    return jnp.maximum(x, 0.0) + jnp.log1p(jnp.exp(-jnp.abs(x)))


def _tc_body(u_ref, t_ref, rel_ref, neg_ref, bias_ref, tri_ref, out_ref, acc_ref):
    i = pl.program_id(0)

    @pl.when(i == 0)
    def _():
        acc_ref[0] = 0.0
        acc_ref[1] = 0.0

    ex = u_ref[...] + rel_ref[0, :][None, :]
    bias = bias_ref[:, 0]
    pos = jnp.sum(ex * t_ref[...], axis=1) + bias
    negl = lax.dot_general(ex, neg_ref[...], (((1,), (1,)), ((), ())),
                           preferred_element_type=jnp.float32) + bias[:, None]
    per_row = _softplus(-pos) + jnp.sum(_softplus(negl), axis=1)
    mask = (tri_ref[:, 1] == 0).astype(jnp.float32)
    acc_ref[0] += jnp.sum(mask * per_row)
    acc_ref[1] += jnp.sum(mask)

    @pl.when(i == _G - 1)
    def _():
        val = acc_ref[0] / jnp.maximum(acc_ref[1], 1.0) / B
        out_ref[...] = jnp.full((1, 1), val, jnp.float32)


def _tc_compute(u_rows, t_rows, purchase_rel, neg_rows, bias3, batch_triples):
    return pl.pallas_call(
        _tc_body,
        grid=(_G,),
        in_specs=[
            pl.BlockSpec((_BLK, EMBED), lambda i: (i, 0)),
            pl.BlockSpec((_BLK, EMBED), lambda i: (i, 0)),
            pl.BlockSpec((1, EMBED), lambda i: (0, 0)),
            pl.BlockSpec((NEG, EMBED), lambda i: (0, 0)),
            pl.BlockSpec((_BLK, 1), lambda i: (i, 0)),
            pl.BlockSpec((_BLK, 3), lambda i: (i, 0)),
        ],
        out_specs=pl.BlockSpec((1, 1), lambda i: (0, 0)),
        out_shape=jax.ShapeDtypeStruct((1, 1), jnp.float32),
        scratch_shapes=[pltpu.SMEM((2,), jnp.float32)],
        compiler_params=pltpu.CompilerParams(
            dimension_semantics=("arbitrary",),
        ),
    )(u_rows, t_rows, purchase_rel, neg_rows, bias3, batch_triples)


def kernel(batch_triples, user_table, product_table, purchase_rel, purchase_bias, neg_idxs):
    u_rows = _sc_user_gather(batch_triples, user_table)
    t_rows, bias3, neg_rows = _sc_prod_gather(
        batch_triples, product_table, purchase_bias, neg_idxs)
    loss = _tc_compute(u_rows, t_rows, purchase_rel, neg_rows, bias3,
                       batch_triples)
    return loss[0, 0]


# speedup vs baseline: 1.0761x; 1.0761x over previous
import jax
import jax.numpy as jnp
from jax import lax
from jax.experimental import pallas as pl
from jax.experimental.pallas import tpu as pltpu
from jax.experimental.pallas import tpu_sc as plsc

EMBED = 64
B = 16384
NEG = 64

_NC = 2
_NS = 16
_NW = _NC * _NS
_BPW = B // _NW
_CH = 128
_K = _BPW // _CH

_G = 8
_BLK = B // _G
_WPG = _BLK // _BPW


def _extract_column(tri_v, col, idx_v):
    lanes = lax.iota(jnp.int32, 16)
    cols = jnp.full((16,), col, jnp.int32)

    def step(k, _):
        rows = lanes + k * 16
        vals = plsc.load_gather(tri_v, [rows, cols])
        idx_v[pl.ds(k * 16, 16)] = vals
        return ()

    lax.fori_loop(0, _BPW // 16, step, ())


def _sc_user_body(tri_hbm, ut_hbm, u_out, tri_v, idx_v, rows_u, sem_u):
    c = lax.axis_index("c")
    s = lax.axis_index("s")
    wid = s * _NC + c
    base = wid * _BPW

    pltpu.sync_copy(tri_hbm.at[pl.ds(base, _BPW), :], tri_v)
    _extract_column(tri_v, 0, idx_v)
    cps = [
        pltpu.async_copy(
            ut_hbm.at[idx_v.at[pl.ds(j * _CH, _CH)]],
            rows_u.at[pl.ds(j * _CH, _CH)], sem_u)
        for j in range(_K)
    ]
    for cp in cps:
        cp.wait()
    pltpu.sync_copy(rows_u, u_out.at[pl.ds(base, _BPW)])


def _sc_user_gather(batch_triples, user_table):
    mesh = plsc.VectorSubcoreMesh(core_axis_name="c", subcore_axis_name="s")
    return pl.kernel(
        _sc_user_body,
        out_type=jax.ShapeDtypeStruct((B, EMBED), jnp.float32),
        mesh=mesh,
        scratch_types=[
            pltpu.VMEM((_BPW, 3), jnp.int32),
            pltpu.VMEM((_BPW,), jnp.int32),
            pltpu.VMEM((_BPW, EMBED), jnp.float32),
            pltpu.SemaphoreType.DMA,
        ],
        compiler_params=pltpu.CompilerParams(use_tc_tiling_on_sc=False,
                                             needs_layout_passes=False),
    )(batch_triples, user_table)


def _sc_prod_body(tri_hbm, pt_hbm, pb_hbm, ni_hbm,
                  t_out, b_out, n_out,
                  tri_v, idx_v, rows_t, bias_v, nidx, nrows,
                  sem_t, sem_b, sem_n):
    c = lax.axis_index("c")
    s = lax.axis_index("s")
    wid = s * _NC + c
    base = wid * _BPW

    pltpu.sync_copy(tri_hbm.at[pl.ds(base, _BPW), :], tri_v)
    _extract_column(tri_v, 2, idx_v)
    cps = []
    for j in range(_K):
        ids = idx_v.at[pl.ds(j * _CH, _CH)]
        cps.append(pltpu.async_copy(
            pt_hbm.at[ids], rows_t.at[pl.ds(j * _CH, _CH)], sem_t))
        cps.append(pltpu.async_copy(
            pb_hbm.at[ids], bias_v.at[pl.ds(j * _CH, _CH)], sem_b))

    @pl.when(wid == 0)
    def _():
        pltpu.sync_copy(ni_hbm, nidx)
        pltpu.async_copy(pt_hbm.at[nidx], nrows, sem_n).wait()
        pltpu.sync_copy(nrows, n_out)

    for cp in cps:
        cp.wait()
    pltpu.sync_copy(rows_t, t_out.at[pl.ds(base, _BPW)])
    pltpu.sync_copy(bias_v, b_out.at[pl.ds(base, _BPW)])


def _sc_prod_gather(batch_triples, product_table, purchase_bias, neg_idxs):
    mesh = plsc.VectorSubcoreMesh(core_axis_name="c", subcore_axis_name="s")
    return pl.kernel(
        _sc_prod_body,
        out_type=(
            jax.ShapeDtypeStruct((B, EMBED), jnp.float32),
            jax.ShapeDtypeStruct((B,), jnp.float32),
            jax.ShapeDtypeStruct((NEG, EMBED), jnp.float32),
        ),
        mesh=mesh,
        scratch_types=[
            pltpu.VMEM((_BPW, 3), jnp.int32),
            pltpu.VMEM((_BPW,), jnp.int32),
            pltpu.VMEM((_BPW, EMBED), jnp.float32),
            pltpu.VMEM((_BPW,), jnp.float32),
            pltpu.VMEM((NEG,), jnp.int32),
            pltpu.VMEM((NEG, EMBED), jnp.float32),
            pltpu.SemaphoreType.DMA,
            pltpu.SemaphoreType.DMA,
            pltpu.SemaphoreType.DMA,
        ],
        compiler_params=pltpu.CompilerParams(use_tc_tiling_on_sc=False,
                                             needs_layout_passes=False),
    )(batch_triples, product_table, purchase_bias.reshape(-1), neg_idxs)


def _softplus(x):
    return jnp.maximum(x, 0.0) + jnp.log1p(jnp.exp(-jnp.abs(x)))


def _tc_body(u_ref, t_ref, rel_ref, neg_ref, bias_ref, tri_ref, out_ref, acc_ref):
    i = pl.program_id(0)

    @pl.when(i == 0)
    def _():
        acc_ref[0] = 0.0
        acc_ref[1] = 0.0

    ex = u_ref[...] + rel_ref[0, :][None, :]
    bias = bias_ref[...]
    pos = jnp.sum(ex * t_ref[...], axis=1) + bias
    negl = lax.dot_general(ex, neg_ref[...], (((1,), (1,)), ((), ())),
                           preferred_element_type=jnp.float32) + bias[:, None]
    per_row = _softplus(-pos) + jnp.sum(_softplus(negl), axis=1)
    mask = (tri_ref[:, 1] == 0).astype(jnp.float32)
    acc_ref[0] += jnp.sum(mask * per_row)
    acc_ref[1] += jnp.sum(mask)

    @pl.when(i == _G - 1)
    def _():
        val = acc_ref[0] / jnp.maximum(acc_ref[1], 1.0) / B
        out_ref[...] = jnp.full((1, 1), val, jnp.float32)


def _tc_compute(u_rows, t_rows, purchase_rel, neg_rows, bias3, batch_triples):
    return pl.pallas_call(
        _tc_body,
        grid=(_G,),
        in_specs=[
            pl.BlockSpec((_BLK, EMBED), lambda i: (i, 0)),
            pl.BlockSpec((_BLK, EMBED), lambda i: (i, 0)),
            pl.BlockSpec((1, EMBED), lambda i: (0, 0)),
            pl.BlockSpec((NEG, EMBED), lambda i: (0, 0)),
            pl.BlockSpec((_BLK,), lambda i: (i,)),
            pl.BlockSpec((_BLK, 3), lambda i: (i, 0)),
        ],
        out_specs=pl.BlockSpec((1, 1), lambda i: (0, 0)),
        out_shape=jax.ShapeDtypeStruct((1, 1), jnp.float32),
        scratch_shapes=[pltpu.SMEM((2,), jnp.float32)],
        compiler_params=pltpu.CompilerParams(
            dimension_semantics=("arbitrary",),
        ),
    )(u_rows, t_rows, purchase_rel, neg_rows, bias3, batch_triples)


def kernel(batch_triples, user_table, product_table, purchase_rel, purchase_bias, neg_idxs):
    u_rows = _sc_user_gather(batch_triples, user_table)
    t_rows, bias3, neg_rows = _sc_prod_gather(
        batch_triples, product_table, purchase_bias, neg_idxs)
    loss = _tc_compute(u_rows, t_rows, purchase_rel, neg_rows, bias3,
                       batch_triples)
    return loss[0, 0]


# speedup vs baseline: 1.0821x; 1.0056x over previous
import jax
import jax.numpy as jnp
from jax import lax
from jax.experimental import pallas as pl
from jax.experimental.pallas import tpu as pltpu
from jax.experimental.pallas import tpu_sc as plsc

EMBED = 64
B = 16384
NEG = 64

_NC = 2
_NS = 16
_NW = _NC * _NS
_BPW = B // _NW
_CH = 128
_K = _BPW // _CH

_G = 8
_BLK = B // _G
_WPG = _BLK // _BPW


def _extract_column(tri_v, col, idx_v):
    lanes = lax.iota(jnp.int32, 16)
    cols = jnp.full((16,), col, jnp.int32)

    def step(k, _):
        rows = lanes + k * 16
        vals = plsc.load_gather(tri_v, [rows, cols])
        idx_v[pl.ds(k * 16, 16)] = vals
        return ()

    lax.fori_loop(0, _BPW // 16, step, ())


def _sc_user_body(h_hbm, ut_hbm, u_out, idx_v, rows_u, sem_u):
    c = lax.axis_index("c")
    s = lax.axis_index("s")
    wid = s * _NC + c
    base = wid * _BPW

    pltpu.sync_copy(h_hbm.at[pl.ds(base, _BPW)], idx_v)
    cps = [
        pltpu.async_copy(
            ut_hbm.at[idx_v.at[pl.ds(j * _CH, _CH)]],
            rows_u.at[pl.ds(j * _CH, _CH)], sem_u)
        for j in range(_K)
    ]
    for cp in cps:
        cp.wait()
    pltpu.sync_copy(rows_u, u_out.at[pl.ds(base, _BPW)])


def _sc_user_gather(h_idxs, user_table):
    mesh = plsc.VectorSubcoreMesh(core_axis_name="c", subcore_axis_name="s")
    return pl.kernel(
        _sc_user_body,
        out_type=jax.ShapeDtypeStruct((B, EMBED), jnp.float32),
        mesh=mesh,
        scratch_types=[
            pltpu.VMEM((_BPW,), jnp.int32),
            pltpu.VMEM((_BPW, EMBED), jnp.float32),
            pltpu.SemaphoreType.DMA,
        ],
        compiler_params=pltpu.CompilerParams(use_tc_tiling_on_sc=False,
                                             needs_layout_passes=False),
    )(h_idxs, user_table)


def _sc_prod_body(t_hbm, pt_hbm, pb_hbm, ni_hbm,
                  t_out, b_out, n_out,
                  idx_v, rows_t, bias_v, nidx, nrows,
                  sem_t, sem_b, sem_n):
    c = lax.axis_index("c")
    s = lax.axis_index("s")
    wid = s * _NC + c
    base = wid * _BPW

    pltpu.sync_copy(t_hbm.at[pl.ds(base, _BPW)], idx_v)
    cps = []
    for j in range(_K):
        ids = idx_v.at[pl.ds(j * _CH, _CH)]
        cps.append(pltpu.async_copy(
            pt_hbm.at[ids], rows_t.at[pl.ds(j * _CH, _CH)], sem_t))
        cps.append(pltpu.async_copy(
            pb_hbm.at[ids], bias_v.at[pl.ds(j * _CH, _CH)], sem_b))

    @pl.when(wid == 0)
    def _():
        pltpu.sync_copy(ni_hbm, nidx)
        pltpu.async_copy(pt_hbm.at[nidx], nrows, sem_n).wait()
        pltpu.sync_copy(nrows, n_out)

    for cp in cps:
        cp.wait()
    pltpu.sync_copy(rows_t, t_out.at[pl.ds(base, _BPW)])
    pltpu.sync_copy(bias_v, b_out.at[pl.ds(base, _BPW)])


def _sc_prod_gather(t_idxs, product_table, purchase_bias, neg_idxs):
    mesh = plsc.VectorSubcoreMesh(core_axis_name="c", subcore_axis_name="s")
    return pl.kernel(
        _sc_prod_body,
        out_type=(
            jax.ShapeDtypeStruct((B, EMBED), jnp.float32),
            jax.ShapeDtypeStruct((B,), jnp.float32),
            jax.ShapeDtypeStruct((NEG, EMBED), jnp.float32),
        ),
        mesh=mesh,
        scratch_types=[
            pltpu.VMEM((_BPW,), jnp.int32),
            pltpu.VMEM((_BPW, EMBED), jnp.float32),
            pltpu.VMEM((_BPW,), jnp.float32),
            pltpu.VMEM((NEG,), jnp.int32),
            pltpu.VMEM((NEG, EMBED), jnp.float32),
            pltpu.SemaphoreType.DMA,
            pltpu.SemaphoreType.DMA,
            pltpu.SemaphoreType.DMA,
        ],
        compiler_params=pltpu.CompilerParams(use_tc_tiling_on_sc=False,
                                             needs_layout_passes=False),
    )(t_idxs, product_table, purchase_bias.reshape(-1), neg_idxs)


def _softplus(x):
    return jnp.maximum(x, 0.0) + jnp.log1p(jnp.exp(-jnp.abs(x)))


def _tc_body(u_ref, t_ref, rel_ref, neg_ref, bias_ref, tri_ref, out_ref, acc_ref):
    i = pl.program_id(0)

    @pl.when(i == 0)
    def _():
        acc_ref[0] = 0.0
        acc_ref[1] = 0.0

    ex = u_ref[...] + rel_ref[0, :][None, :]
    bias = bias_ref[...]
    pos = jnp.sum(ex * t_ref[...], axis=1) + bias
    negl = lax.dot_general(ex, neg_ref[...], (((1,), (1,)), ((), ())),
                           preferred_element_type=jnp.float32) + bias[:, None]
    per_row = _softplus(-pos) + jnp.sum(_softplus(negl), axis=1)
    mask = (tri_ref[:, 1] == 0).astype(jnp.float32)
    acc_ref[0] += jnp.sum(mask * per_row)
    acc_ref[1] += jnp.sum(mask)

    @pl.when(i == _G - 1)
    def _():
        val = acc_ref[0] / jnp.maximum(acc_ref[1], 1.0) / B
        out_ref[...] = jnp.full((1, 1), val, jnp.float32)


def _tc_compute(u_rows, t_rows, purchase_rel, neg_rows, bias3, batch_triples):
    return pl.pallas_call(
        _tc_body,
        grid=(_G,),
        in_specs=[
            pl.BlockSpec((_BLK, EMBED), lambda i: (i, 0)),
            pl.BlockSpec((_BLK, EMBED), lambda i: (i, 0)),
            pl.BlockSpec((1, EMBED), lambda i: (0, 0)),
            pl.BlockSpec((NEG, EMBED), lambda i: (0, 0)),
            pl.BlockSpec((_BLK,), lambda i: (i,)),
            pl.BlockSpec((_BLK, 3), lambda i: (i, 0)),
        ],
        out_specs=pl.BlockSpec((1, 1), lambda i: (0, 0)),
        out_shape=jax.ShapeDtypeStruct((1, 1), jnp.float32),
        scratch_shapes=[pltpu.SMEM((2,), jnp.float32)],
        compiler_params=pltpu.CompilerParams(
            dimension_semantics=("arbitrary",),
        ),
    )(u_rows, t_rows, purchase_rel, neg_rows, bias3, batch_triples)


def kernel(batch_triples, user_table, product_table, purchase_rel, purchase_bias, neg_idxs):
    u_rows = _sc_user_gather(batch_triples[:, 0], user_table)
    t_rows, bias3, neg_rows = _sc_prod_gather(
        batch_triples[:, 2], product_table, purchase_bias, neg_idxs)
    loss = _tc_compute(u_rows, t_rows, purchase_rel, neg_rows, bias3,
                       batch_triples)
    return loss[0, 0]
